# baseline (speedup 1.0000x reference)
import functools

import jax
import jax.numpy as jnp
from jax import lax
from jax.experimental import pallas as pl
from jax.experimental.pallas import tpu as pltpu
from jax.experimental.pallas import tpu_sc as plsc

N = 10000
E = 320000
B = 64
D = 128
H = 32

NP = 10240
ROWS_PER_TILE = NP // 16
CHUNK = 128
NCHUNKS = 2560
EP = NCHUNKS * CHUNK
CA = 88
CB = 72
KG = 8
NGA = CA // KG
NGB = CB // KG
RB = 1024
NBLK = NP // RB


def _mesh():
    return plsc.VectorSubcoreMesh(core_axis_name="c", subcore_axis_name="s")


def _per_core_span(c, s):
    nc = jnp.where(c == 0, CA, CB)
    base = jnp.where(c == 0, s * CA, 16 * CA + s * CB)
    ng = jnp.where(c == 0, NGA, NGB)
    return nc, base, ng


EC = E // CHUNK
DCA = 112
DCB = 44


def _sc_degree(dstc):
    @functools.partial(
        pl.kernel,
        out_type=jax.ShapeDtypeStruct((2, NP), jnp.float32),
        mesh=_mesh(),
        compiler_params=pltpu.CompilerParams(use_tc_tiling_on_sc=False),
        scratch_types=[
            pltpu.VMEM((DCA, CHUNK), jnp.int32),
            pltpu.VMEM((CHUNK,), jnp.float32),
            pltpu.VMEM((ROWS_PER_TILE,), jnp.float32),
            pltpu.VMEM_SHARED((NP,), jnp.float32),
            pltpu.SemaphoreType.DMA,
        ],
    )
    def k(dstc_hbm, out_hbm, didx, ones_v, ibuf, dacc, sem):
        c = lax.axis_index("c")
        s = lax.axis_index("s")
        nc = jnp.where(c == 0, DCA, DCB + (s >= 12).astype(jnp.int32))
        base = jnp.where(c == 0, s * DCA,
                         16 * DCA + s * DCB + jnp.maximum(s - 12, 0))

        @pl.when(c == 0)
        def _():
            pltpu.sync_copy(dstc_hbm.at[pl.ds(base, DCA)], didx)

        @pl.when(c == 1)
        def _():
            pltpu.sync_copy(dstc_hbm.at[pl.ds(base, DCB + 1)],
                            didx.at[pl.ds(0, DCB + 1)])

        init = jnp.where(c == 0, 1.0, 0.0).astype(jnp.float32)

        def fill_i(r, carry):
            ibuf[pl.ds(r * 16, 16)] = jnp.zeros((16,), jnp.float32) + init
            return carry

        lax.fori_loop(0, ROWS_PER_TILE // 16, fill_i, 0)

        def fill_o(r, carry):
            ones_v[pl.ds(r * 16, 16)] = jnp.ones((16,), jnp.float32)
            return carry

        lax.fori_loop(0, CHUNK // 16, fill_o, 0)

        pltpu.sync_copy(ibuf, dacc.at[pl.ds(s * ROWS_PER_TILE, ROWS_PER_TILE)])
        plsc.subcore_barrier()

        def fire(g, carry):
            pltpu.async_copy(ones_v, dacc.at[didx.at[g]], sem, add=True)
            return carry

        lax.fori_loop(0, nc, fire, 0)

        def drain(g, carry):
            pltpu.make_async_copy(ones_v, dacc.at[didx.at[g]], sem).wait()
            return carry

        lax.fori_loop(0, nc, drain, 0)
        plsc.subcore_barrier()
        pltpu.sync_copy(dacc.at[pl.ds(s * ROWS_PER_TILE, ROWS_PER_TILE)],
                        out_hbm.at[c, pl.ds(s * ROWS_PER_TILE, ROWS_PER_TILE)])

    return k(dstc)


def _sc_edge_agg(y, srcp, dstp):
    @functools.partial(
        pl.kernel,
        out_type=jax.ShapeDtypeStruct((2, NP, H), jnp.float32),
        mesh=_mesh(),
        compiler_params=pltpu.CompilerParams(use_tc_tiling_on_sc=False),
        scratch_types=[
            pltpu.VMEM((CA, CHUNK), jnp.int32),
            pltpu.VMEM((CA, CHUNK), jnp.int32),
            pltpu.VMEM((KG * CHUNK, H), jnp.float32),
            pltpu.VMEM((KG * CHUNK, H), jnp.float32),
            pltpu.VMEM_SHARED((NP, H), jnp.float32),
            pltpu.VMEM_SHARED((NP, H), jnp.float32),
            pltpu.SemaphoreType.DMA,
            pltpu.SemaphoreType.DMA,
            pltpu.SemaphoreType.DMA,
            pltpu.SemaphoreType.DMA,
        ],
    )
    def k(y_hbm, srcp_hbm, dstp_hbm, out_hbm, sidx, didx, buf_a, buf_b,
          acc, ycache, sem_ga, sem_gb, sem_sa, sem_sb):
        c = lax.axis_index("c")
        s = lax.axis_index("s")
        _, base, ng = _per_core_span(c, s)

        row0 = s * ROWS_PER_TILE
        pltpu.async_copy(y_hbm.at[pl.ds(row0, ROWS_PER_TILE)],
                         ycache.at[pl.ds(row0, ROWS_PER_TILE)], sem_sa)

        @pl.when(c == 0)
        def _():
            pltpu.async_copy(srcp_hbm.at[pl.ds(base, CA)], sidx, sem_ga)
            pltpu.async_copy(dstp_hbm.at[pl.ds(base, CA)], didx, sem_gb)

        @pl.when(c == 1)
        def _():
            pltpu.async_copy(srcp_hbm.at[pl.ds(base, CB)],
                             sidx.at[pl.ds(0, CB)], sem_ga)
            pltpu.async_copy(dstp_hbm.at[pl.ds(base, CB)],
                             didx.at[pl.ds(0, CB)], sem_gb)

        def zr(r, carry):
            buf_a[r, pl.ds(0, 16)] = jnp.zeros((16,), jnp.float32)
            buf_a[r, pl.ds(16, 16)] = jnp.zeros((16,), jnp.float32)
            return carry

        lax.fori_loop(0, CHUNK, zr, 0)
        for rep in range(ROWS_PER_TILE // CHUNK):
            pltpu.sync_copy(
                buf_a.at[pl.ds(0, CHUNK)],
                acc.at[pl.ds(s * ROWS_PER_TILE + rep * CHUNK, CHUNK)])

        @pl.when(c == 0)
        def _():
            pltpu.make_async_copy(srcp_hbm.at[pl.ds(base, CA)],
                                  sidx, sem_ga).wait()
            pltpu.make_async_copy(dstp_hbm.at[pl.ds(base, CA)],
                                  didx, sem_gb).wait()

        @pl.when(c == 1)
        def _():
            pltpu.make_async_copy(srcp_hbm.at[pl.ds(base, CB)],
                                  sidx.at[pl.ds(0, CB)], sem_ga).wait()
            pltpu.make_async_copy(dstp_hbm.at[pl.ds(base, CB)],
                                  didx.at[pl.ds(0, CB)], sem_gb).wait()

        pltpu.make_async_copy(y_hbm.at[pl.ds(row0, ROWS_PER_TILE)],
                              ycache.at[pl.ds(row0, ROWS_PER_TILE)],
                              sem_sa).wait()
        plsc.subcore_barrier()

        def fire_g(grp, buf, sem):
            for j in range(KG):
                pltpu.async_copy(ycache.at[sidx.at[grp * KG + j]],
                                 buf.at[pl.ds(j * CHUNK, CHUNK)], sem)

        def drain_g(grp, buf, sem):
            for j in range(KG):
                pltpu.make_async_copy(ycache.at[sidx.at[grp * KG + j]],
                                      buf.at[pl.ds(j * CHUNK, CHUNK)],
                                      sem).wait()

        def fire_s(grp, buf, sem):
            for j in range(KG):
                pltpu.async_copy(buf.at[pl.ds(j * CHUNK, CHUNK)],
                                 acc.at[didx.at[grp * KG + j]], sem, add=True)

        def drain_s(grp, buf, sem):
            for j in range(KG):
                pltpu.make_async_copy(buf.at[pl.ds(j * CHUNK, CHUNK)],
                                      acc.at[didx.at[grp * KG + j]],
                                      sem).wait()

        fire_g(0, buf_a, sem_ga)
        fire_g(1, buf_b, sem_gb)

        def group_body(t, buf, sem_g, sem_s):
            drain_g(t, buf, sem_g)
            fire_s(t, buf, sem_s)

            @pl.when(t + 2 < ng)
            def _():
                drain_s(t, buf, sem_s)
                fire_g(t + 2, buf, sem_g)

        def step(t, carry):
            @pl.when(t % 2 == 0)
            def _():
                group_body(t, buf_a, sem_ga, sem_sa)

            @pl.when(t % 2 == 1)
            def _():
                group_body(t, buf_b, sem_gb, sem_sb)

            return carry

        lax.fori_loop(0, ng, step, 0)
        drain_s(ng - 2, buf_b, sem_sb)
        drain_s(ng - 1, buf_a, sem_sa)
        plsc.subcore_barrier()
        pltpu.sync_copy(acc.at[pl.ds(s * ROWS_PER_TILE, ROWS_PER_TILE)],
                        out_hbm.at[c, pl.ds(s * ROWS_PER_TILE, ROWS_PER_TILE)])

    return k(y, srcp, dstp)


def _tc_mm(x_p, W):
    def body(x_ref, w_ref, o_ref):
        o_ref[...] = jnp.dot(x_ref[...], w_ref[...],
                             preferred_element_type=jnp.float32)

    din = x_p.shape[1]
    return pl.pallas_call(
        body,
        grid=(NBLK,),
        in_specs=[
            pl.BlockSpec((RB, din), lambda i: (i, 0)),
            pl.BlockSpec((din, H), lambda i: (0, 0)),
        ],
        out_specs=pl.BlockSpec((RB, H), lambda i: (i, 0)),
        out_shape=jax.ShapeDtypeStruct((NP, H), jnp.float32),
    )(x_p, W)


def _tc_scale(u, degp):
    def body(u_ref, d_ref, o_ref):
        dinv = lax.rsqrt(d_ref[0] + d_ref[1])
        o_ref[...] = dinv * u_ref[...]

    return pl.pallas_call(
        body,
        grid=(NBLK,),
        in_specs=[
            pl.BlockSpec((RB, H), lambda i: (i, 0)),
            pl.BlockSpec((2, RB, 1), lambda i: (0, i, 0)),
        ],
        out_specs=pl.BlockSpec((RB, H), lambda i: (i, 0)),
        out_shape=jax.ShapeDtypeStruct((NP, H), jnp.float32),
    )(u, degp)


def _tc_layer_mid(accp, y1, degp, b1, W2):
    def body(a_ref, y_ref, d_ref, b_ref, w_ref, o_ref):
        dinv = lax.rsqrt(d_ref[0] + d_ref[1])
        agg = a_ref[0] + a_ref[1] + y_ref[...]
        h1 = jnp.tanh(dinv * agg + b_ref[...])
        o_ref[...] = dinv * jnp.dot(h1, w_ref[...],
                                    preferred_element_type=jnp.float32)

    return pl.pallas_call(
        body,
        grid=(NBLK,),
        in_specs=[
            pl.BlockSpec((2, RB, H), lambda i: (0, i, 0)),
            pl.BlockSpec((RB, H), lambda i: (i, 0)),
            pl.BlockSpec((2, RB, 1), lambda i: (0, i, 0)),
            pl.BlockSpec((1, H), lambda i: (0, 0)),
            pl.BlockSpec((H, H), lambda i: (0, 0)),
        ],
        out_specs=pl.BlockSpec((RB, H), lambda i: (i, 0)),
        out_shape=jax.ShapeDtypeStruct((NP, H), jnp.float32),
    )(accp, y1, degp, b1, W2)


def _tc_final(accp, y2, degp, vb, b2, W3, b3, W4, b4, ss,
              Wv1, bv1, Wv2, bv2, Wv3, bv3, Wc1, bc1, Wc2, bc2):
    def body(a_ref, y_ref, d_ref, vb_ref, b2_ref, w3_ref, b3_ref, w4_ref,
             b4_ref, ss_ref, wv1_ref, bv1_ref, wv2_ref, bv2_ref, wv3_ref,
             bv3_ref, wc1_ref, bc1_ref, wc2_ref, bc2_ref, o_ref, g_ref):
        i = pl.program_id(0)
        dinv = lax.rsqrt(d_ref[0] + d_ref[1])
        agg = a_ref[0] + a_ref[1] + y_ref[...]
        h2 = jnp.tanh(dinv * agg + b2_ref[...])
        h3 = jnp.tanh(jnp.dot(h2, w3_ref[...],
                              preferred_element_type=jnp.float32) + b3_ref[...])
        seg = lax.broadcasted_iota(jnp.int32, (1, B), 1)
        pb = (vb_ref[...] == seg).astype(jnp.float32)
        part = lax.dot_general(pb, h3, (((0,), (0,)), ((), ())),
                               preferred_element_type=jnp.float32)

        @pl.when(i == 0)
        def _():
            g_ref[...] = part

        @pl.when(i > 0)
        def _():
            g_ref[...] = g_ref[...] + part

        @pl.when(i == NBLK - 1)
        def _():
            g = g_ref[...]
            h1v = jnp.dot(g, w4_ref[...],
                          preferred_element_type=jnp.float32) + b4_ref[...]
            t = jnp.tanh(jnp.dot(ss_ref[...], wv1_ref[...],
                                 preferred_element_type=jnp.float32) + bv1_ref[...])
            t = jnp.tanh(jnp.dot(t, wv2_ref[...],
                                 preferred_element_type=jnp.float32) + bv2_ref[...])
            h2v = jnp.dot(t, wv3_ref[...],
                          preferred_element_type=jnp.float32) + bv3_ref[...]
            cc = jnp.tanh(jnp.dot(h1v, wc1_ref[pl.ds(0, H), :],
                                  preferred_element_type=jnp.float32)
                          + jnp.dot(h2v, wc1_ref[pl.ds(H, H), :],
                                    preferred_element_type=jnp.float32)
                          + bc1_ref[...])
            o_ref[...] = jnp.dot(cc, wc2_ref[...],
                                 preferred_element_type=jnp.float32) + bc2_ref[...]

    full = lambda shp: pl.BlockSpec(shp, lambda i: tuple(0 for _ in shp))
    return pl.pallas_call(
        body,
        grid=(NBLK,),
        in_specs=[
            pl.BlockSpec((2, RB, H), lambda i: (0, i, 0)),
            pl.BlockSpec((RB, H), lambda i: (i, 0)),
            pl.BlockSpec((2, RB, 1), lambda i: (0, i, 0)),
            pl.BlockSpec((RB, 1), lambda i: (i, 0)),
            full((1, H)), full((H, H)), full((1, H)), full((H, H)),
            full((1, H)), full((B, D)), full((D, H)), full((1, H)),
            full((H, H)), full((1, H)), full((H, H)), full((1, H)),
            full((2 * H, H)), full((1, H)), full((H, 1)), full((1, 1)),
        ],
        out_specs=pl.BlockSpec((B, 1), lambda i: (0, 0)),
        out_shape=jax.ShapeDtypeStruct((B, 1), jnp.float32),
        scratch_shapes=[pltpu.VMEM((B, H), jnp.float32)],
    )(accp, y2, degp, vb, b2, W3, b3, W4, b4, ss,
      Wv1, bv1, Wv2, bv2, Wv3, bv3, Wc1, bc1, Wc2, bc2)


def kernel(x, edge_index, share_state, value_batch, W1, b1, W2, b2, W3, b3,
           W4, b4, Wv1, bv1, Wv2, bv2, Wv3, bv3, Wc1, bc1, Wc2, bc2):
    f32 = jnp.float32
    i32 = jnp.int32

    x_p = jnp.concatenate([x, jnp.zeros((NP - N, D), f32)], axis=0)
    src = edge_index[0].astype(i32)
    dst = edge_index[1].astype(i32)
    srcp = jnp.concatenate([src, jnp.zeros((EP - E,), i32)]).reshape(
        NCHUNKS, CHUNK)
    dstp = jnp.concatenate([dst, jnp.full((EP - E,), N, i32)]).reshape(
        NCHUNKS, CHUNK)
    vb = jnp.concatenate([value_batch.astype(i32),
                          jnp.full((NP - N,), B, i32)]).reshape(NP, 1)

    dstc = dst.reshape(EC, CHUNK)
    degp = _sc_degree(dstc).reshape(2, NP, 1)

    u1 = _tc_mm(x_p, W1)
    y1 = _tc_scale(u1, degp)
    acc1 = _sc_edge_agg(y1, srcp, dstp)
    y2 = _tc_layer_mid(acc1, y1, degp, b1.reshape(1, H), W2)
    acc2 = _sc_edge_agg(y2, srcp, dstp)
    value = _tc_final(acc2, y2, degp, vb, b2.reshape(1, H), W3,
                      b3.reshape(1, H), W4, b4.reshape(1, H), share_state,
                      Wv1, bv1.reshape(1, H), Wv2, bv2.reshape(1, H), Wv3,
                      bv3.reshape(1, H), Wc1, bc1.reshape(1, H), Wc2,
                      bc2.reshape(1, 1))
    return value


# speedup vs baseline: 1.0474x; 1.0474x over previous
import functools

import jax
import jax.numpy as jnp
from jax import lax
from jax.experimental import pallas as pl
from jax.experimental.pallas import tpu as pltpu
from jax.experimental.pallas import tpu_sc as plsc

N = 10000
E = 320000
B = 64
D = 128
H = 32

NP = 10240
ROWS_PER_TILE = NP // 16
CHUNK = 128
NCHUNKS = 2560
EP = NCHUNKS * CHUNK
CA = 88
CB = 72
KG = 8
NGA = CA // KG
NGB = CB // KG
RB = 1024
NBLK = NP // RB


def _mesh():
    return plsc.VectorSubcoreMesh(core_axis_name="c", subcore_axis_name="s")


def _per_core_span(c, s):
    nc = jnp.where(c == 0, CA, CB)
    base = jnp.where(c == 0, s * CA, 16 * CA + s * CB)
    ng = jnp.where(c == 0, NGA, NGB)
    return nc, base, ng


EC = E // CHUNK
DCA = 112
DCB = 44


def _sc_degree(dstc):
    @functools.partial(
        pl.kernel,
        out_type=jax.ShapeDtypeStruct((2, NP), jnp.float32),
        mesh=_mesh(),
        compiler_params=pltpu.CompilerParams(use_tc_tiling_on_sc=False),
        scratch_types=[
            pltpu.VMEM((DCA, CHUNK), jnp.int32),
            pltpu.VMEM((CHUNK,), jnp.float32),
            pltpu.VMEM((ROWS_PER_TILE,), jnp.float32),
            pltpu.VMEM_SHARED((NP,), jnp.float32),
            pltpu.SemaphoreType.DMA,
        ],
    )
    def k(dstc_hbm, out_hbm, didx, ones_v, ibuf, dacc, sem):
        c = lax.axis_index("c")
        s = lax.axis_index("s")
        nc = jnp.where(c == 0, DCA, DCB + (s >= 12).astype(jnp.int32))
        base = jnp.where(c == 0, s * DCA,
                         16 * DCA + s * DCB + jnp.maximum(s - 12, 0))

        @pl.when(c == 0)
        def _():
            pltpu.sync_copy(dstc_hbm.at[pl.ds(base, DCA)], didx)

        @pl.when(c == 1)
        def _():
            pltpu.sync_copy(dstc_hbm.at[pl.ds(base, DCB + 1)],
                            didx.at[pl.ds(0, DCB + 1)])

        init = jnp.where(c == 0, 1.0, 0.0).astype(jnp.float32)

        def fill_i(r, carry):
            ibuf[pl.ds(r * 16, 16)] = jnp.zeros((16,), jnp.float32) + init
            return carry

        lax.fori_loop(0, ROWS_PER_TILE // 16, fill_i, 0)

        def fill_o(r, carry):
            ones_v[pl.ds(r * 16, 16)] = jnp.ones((16,), jnp.float32)
            return carry

        lax.fori_loop(0, CHUNK // 16, fill_o, 0)

        pltpu.sync_copy(ibuf, dacc.at[pl.ds(s * ROWS_PER_TILE, ROWS_PER_TILE)])
        plsc.subcore_barrier()

        def fire(g, carry):
            pltpu.async_copy(ones_v, dacc.at[didx.at[g]], sem, add=True)
            return carry

        lax.fori_loop(0, nc, fire, 0)

        def drain(g, carry):
            pltpu.make_async_copy(ones_v, dacc.at[didx.at[g]], sem).wait()
            return carry

        lax.fori_loop(0, nc, drain, 0)
        plsc.subcore_barrier()
        pltpu.sync_copy(dacc.at[pl.ds(s * ROWS_PER_TILE, ROWS_PER_TILE)],
                        out_hbm.at[c, pl.ds(s * ROWS_PER_TILE, ROWS_PER_TILE)])

    return k(dstc)


def _sc_edge_agg(y, srcp, dstp):
    @functools.partial(
        pl.kernel,
        out_type=jax.ShapeDtypeStruct((2, NP, H), jnp.float32),
        mesh=_mesh(),
        compiler_params=pltpu.CompilerParams(use_tc_tiling_on_sc=False),
        scratch_types=[
            pltpu.VMEM((CA, CHUNK), jnp.int32),
            pltpu.VMEM((CA, CHUNK), jnp.int32),
            pltpu.VMEM((KG * CHUNK, H), jnp.float32),
            pltpu.VMEM((KG * CHUNK, H), jnp.float32),
            pltpu.VMEM_SHARED((NP, H), jnp.float32),
            pltpu.VMEM_SHARED((NP, H), jnp.float32),
            pltpu.SemaphoreType.DMA,
            pltpu.SemaphoreType.DMA,
            pltpu.SemaphoreType.DMA,
            pltpu.SemaphoreType.DMA,
        ],
    )
    def k(y_hbm, srcp_hbm, dstp_hbm, out_hbm, sidx, didx, buf_a, buf_b,
          acc, ycache, sem_ga, sem_gb, sem_sa, sem_sb):
        c = lax.axis_index("c")
        s = lax.axis_index("s")
        _, base, ng = _per_core_span(c, s)

        row0 = s * ROWS_PER_TILE
        pltpu.async_copy(y_hbm.at[pl.ds(row0, ROWS_PER_TILE)],
                         ycache.at[pl.ds(row0, ROWS_PER_TILE)], sem_sa)

        @pl.when(c == 0)
        def _():
            pltpu.async_copy(srcp_hbm.at[pl.ds(base, CA)], sidx, sem_ga)
            pltpu.async_copy(dstp_hbm.at[pl.ds(base, CA)], didx, sem_gb)

        @pl.when(c == 1)
        def _():
            pltpu.async_copy(srcp_hbm.at[pl.ds(base, CB)],
                             sidx.at[pl.ds(0, CB)], sem_ga)
            pltpu.async_copy(dstp_hbm.at[pl.ds(base, CB)],
                             didx.at[pl.ds(0, CB)], sem_gb)

        def zr(r, carry):
            buf_a[r, pl.ds(0, 16)] = jnp.zeros((16,), jnp.float32)
            buf_a[r, pl.ds(16, 16)] = jnp.zeros((16,), jnp.float32)
            return carry

        lax.fori_loop(0, CHUNK, zr, 0)
        for rep in range(ROWS_PER_TILE // CHUNK):
            pltpu.sync_copy(
                buf_a.at[pl.ds(0, CHUNK)],
                acc.at[pl.ds(s * ROWS_PER_TILE + rep * CHUNK, CHUNK)])

        @pl.when(c == 0)
        def _():
            pltpu.make_async_copy(srcp_hbm.at[pl.ds(base, CA)],
                                  sidx, sem_ga).wait()
            pltpu.make_async_copy(dstp_hbm.at[pl.ds(base, CA)],
                                  didx, sem_gb).wait()

        @pl.when(c == 1)
        def _():
            pltpu.make_async_copy(srcp_hbm.at[pl.ds(base, CB)],
                                  sidx.at[pl.ds(0, CB)], sem_ga).wait()
            pltpu.make_async_copy(dstp_hbm.at[pl.ds(base, CB)],
                                  didx.at[pl.ds(0, CB)], sem_gb).wait()

        pltpu.make_async_copy(y_hbm.at[pl.ds(row0, ROWS_PER_TILE)],
                              ycache.at[pl.ds(row0, ROWS_PER_TILE)],
                              sem_sa).wait()
        plsc.subcore_barrier()

        def fire_g(grp, buf, sem):
            for j in range(KG):
                pltpu.async_copy(ycache.at[sidx.at[grp * KG + j]],
                                 buf.at[pl.ds(j * CHUNK, CHUNK)], sem)

        def drain_g(grp, buf, sem):
            for j in range(KG):
                pltpu.make_async_copy(ycache.at[sidx.at[grp * KG + j]],
                                      buf.at[pl.ds(j * CHUNK, CHUNK)],
                                      sem).wait()

        def fire_s(grp, buf, sem):
            for j in range(KG):
                pltpu.async_copy(buf.at[pl.ds(j * CHUNK, CHUNK)],
                                 acc.at[didx.at[grp * KG + j]], sem, add=True)

        def drain_s(grp, buf, sem):
            for j in range(KG):
                pltpu.make_async_copy(buf.at[pl.ds(j * CHUNK, CHUNK)],
                                      acc.at[didx.at[grp * KG + j]],
                                      sem).wait()

        fire_g(0, buf_a, sem_ga)
        fire_g(1, buf_b, sem_gb)

        def group_body(t, buf, sem_g, sem_s):
            drain_g(t, buf, sem_g)
            fire_s(t, buf, sem_s)

            @pl.when(t + 2 < ng)
            def _():
                drain_s(t, buf, sem_s)
                fire_g(t + 2, buf, sem_g)

        def step(t, carry):
            @pl.when(t % 2 == 0)
            def _():
                group_body(t, buf_a, sem_ga, sem_sa)

            @pl.when(t % 2 == 1)
            def _():
                group_body(t, buf_b, sem_gb, sem_sb)

            return carry

        lax.fori_loop(0, ng, step, 0)
        drain_s(ng - 2, buf_b, sem_sb)
        drain_s(ng - 1, buf_a, sem_sa)
        plsc.subcore_barrier()
        pltpu.sync_copy(acc.at[pl.ds(s * ROWS_PER_TILE, ROWS_PER_TILE)],
                        out_hbm.at[c, pl.ds(s * ROWS_PER_TILE, ROWS_PER_TILE)])

    return k(y, srcp, dstp)


def _tc_scale_mm(x_p, degp, W):
    def body(x_ref, d_ref, w_ref, o_ref):
        dinv = lax.rsqrt(d_ref[0] + d_ref[1])
        o_ref[...] = dinv * jnp.dot(x_ref[...], w_ref[...],
                                    preferred_element_type=jnp.float32)

    din = x_p.shape[1]
    return pl.pallas_call(
        body,
        grid=(NBLK,),
        in_specs=[
            pl.BlockSpec((RB, din), lambda i: (i, 0)),
            pl.BlockSpec((2, RB, 1), lambda i: (0, i, 0)),
            pl.BlockSpec((din, H), lambda i: (0, 0)),
        ],
        out_specs=pl.BlockSpec((RB, H), lambda i: (i, 0)),
        out_shape=jax.ShapeDtypeStruct((NP, H), jnp.float32),
    )(x_p, degp, W)


def _tc_layer_mid(accp, y1, degp, b1, W2):
    def body(a_ref, y_ref, d_ref, b_ref, w_ref, o_ref):
        dinv = lax.rsqrt(d_ref[0] + d_ref[1])
        agg = a_ref[0] + a_ref[1] + y_ref[...]
        h1 = jnp.tanh(dinv * agg + b_ref[...])
        o_ref[...] = dinv * jnp.dot(h1, w_ref[...],
                                    preferred_element_type=jnp.float32)

    return pl.pallas_call(
        body,
        grid=(NBLK,),
        in_specs=[
            pl.BlockSpec((2, RB, H), lambda i: (0, i, 0)),
            pl.BlockSpec((RB, H), lambda i: (i, 0)),
            pl.BlockSpec((2, RB, 1), lambda i: (0, i, 0)),
            pl.BlockSpec((1, H), lambda i: (0, 0)),
            pl.BlockSpec((H, H), lambda i: (0, 0)),
        ],
        out_specs=pl.BlockSpec((RB, H), lambda i: (i, 0)),
        out_shape=jax.ShapeDtypeStruct((NP, H), jnp.float32),
    )(accp, y1, degp, b1, W2)


def _tc_final(accp, y2, degp, vb, b2, W3, b3, W4, b4, ss,
              Wv1, bv1, Wv2, bv2, Wv3, bv3, Wc1, bc1, Wc2, bc2):
    def body(a_ref, y_ref, d_ref, vb_ref, b2_ref, w3_ref, b3_ref, w4_ref,
             b4_ref, ss_ref, wv1_ref, bv1_ref, wv2_ref, bv2_ref, wv3_ref,
             bv3_ref, wc1_ref, bc1_ref, wc2_ref, bc2_ref, o_ref, g_ref):
        i = pl.program_id(0)
        dinv = lax.rsqrt(d_ref[0] + d_ref[1])
        agg = a_ref[0] + a_ref[1] + y_ref[...]
        h2 = jnp.tanh(dinv * agg + b2_ref[...])
        h3 = jnp.tanh(jnp.dot(h2, w3_ref[...],
                              preferred_element_type=jnp.float32) + b3_ref[...])
        seg = lax.broadcasted_iota(jnp.int32, (1, B), 1)
        pb = (vb_ref[...] == seg).astype(jnp.float32)
        part = lax.dot_general(pb, h3, (((0,), (0,)), ((), ())),
                               preferred_element_type=jnp.float32)

        @pl.when(i == 0)
        def _():
            g_ref[...] = part

        @pl.when(i > 0)
        def _():
            g_ref[...] = g_ref[...] + part

        @pl.when(i == NBLK - 1)
        def _():
            g = g_ref[...]
            h1v = jnp.dot(g, w4_ref[...],
                          preferred_element_type=jnp.float32) + b4_ref[...]
            t = jnp.tanh(jnp.dot(ss_ref[...], wv1_ref[...],
                                 preferred_element_type=jnp.float32) + bv1_ref[...])
            t = jnp.tanh(jnp.dot(t, wv2_ref[...],
                                 preferred_element_type=jnp.float32) + bv2_ref[...])
            h2v = jnp.dot(t, wv3_ref[...],
                          preferred_element_type=jnp.float32) + bv3_ref[...]
            cc = jnp.tanh(jnp.dot(h1v, wc1_ref[pl.ds(0, H), :],
                                  preferred_element_type=jnp.float32)
                          + jnp.dot(h2v, wc1_ref[pl.ds(H, H), :],
                                    preferred_element_type=jnp.float32)
                          + bc1_ref[...])
            o_ref[...] = jnp.dot(cc, wc2_ref[...],
                                 preferred_element_type=jnp.float32) + bc2_ref[...]

    full = lambda shp: pl.BlockSpec(shp, lambda i: tuple(0 for _ in shp))
    return pl.pallas_call(
        body,
        grid=(NBLK,),
        in_specs=[
            pl.BlockSpec((2, RB, H), lambda i: (0, i, 0)),
            pl.BlockSpec((RB, H), lambda i: (i, 0)),
            pl.BlockSpec((2, RB, 1), lambda i: (0, i, 0)),
            pl.BlockSpec((RB, 1), lambda i: (i, 0)),
            full((1, H)), full((H, H)), full((1, H)), full((H, H)),
            full((1, H)), full((B, D)), full((D, H)), full((1, H)),
            full((H, H)), full((1, H)), full((H, H)), full((1, H)),
            full((2 * H, H)), full((1, H)), full((H, 1)), full((1, 1)),
        ],
        out_specs=pl.BlockSpec((B, 1), lambda i: (0, 0)),
        out_shape=jax.ShapeDtypeStruct((B, 1), jnp.float32),
        scratch_shapes=[pltpu.VMEM((B, H), jnp.float32)],
    )(accp, y2, degp, vb, b2, W3, b3, W4, b4, ss,
      Wv1, bv1, Wv2, bv2, Wv3, bv3, Wc1, bc1, Wc2, bc2)


def kernel(x, edge_index, share_state, value_batch, W1, b1, W2, b2, W3, b3,
           W4, b4, Wv1, bv1, Wv2, bv2, Wv3, bv3, Wc1, bc1, Wc2, bc2):
    f32 = jnp.float32
    i32 = jnp.int32

    x_p = jnp.concatenate([x, jnp.zeros((NP - N, D), f32)], axis=0)
    src = edge_index[0].astype(i32)
    dst = edge_index[1].astype(i32)
    srcp = jnp.concatenate([src, jnp.zeros((EP - E,), i32)]).reshape(
        NCHUNKS, CHUNK)
    dstp = jnp.concatenate([dst, jnp.full((EP - E,), N, i32)]).reshape(
        NCHUNKS, CHUNK)
    vb = jnp.concatenate([value_batch.astype(i32),
                          jnp.full((NP - N,), B, i32)]).reshape(NP, 1)

    dstc = dst.reshape(EC, CHUNK)
    degp = _sc_degree(dstc).reshape(2, NP, 1)

    y1 = _tc_scale_mm(x_p, degp, W1)
    acc1 = _sc_edge_agg(y1, srcp, dstp)
    y2 = _tc_layer_mid(acc1, y1, degp, b1.reshape(1, H), W2)
    acc2 = _sc_edge_agg(y2, srcp, dstp)
    value = _tc_final(acc2, y2, degp, vb, b2.reshape(1, H), W3,
                      b3.reshape(1, H), W4, b4.reshape(1, H), share_state,
                      Wv1, bv1.reshape(1, H), Wv2, bv2.reshape(1, H), Wv3,
                      bv3.reshape(1, H), Wc1, bc1.reshape(1, H), Wc2,
                      bc2.reshape(1, 1))
    return value
